# baseline (speedup 1.0000x reference)
import functools

import jax
import jax.numpy as jnp
from jax import lax
from jax.experimental import pallas as pl
from jax.experimental.pallas import tpu as pltpu
from jax.experimental.pallas import tpu_sc as plsc

VOCAB = 100000
EMBED = 64
BATCH = 1024
CTX = 50

NC = 2
NS = 16
NW = NC * NS
NB = BATCH // NW
NIDX = NB * CTX

_sc_mesh = plsc.VectorSubcoreMesh(core_axis_name="c", subcore_axis_name="s")


@functools.partial(
    pl.kernel,
    mesh=_sc_mesh,
    out_type=jax.ShapeDtypeStruct((BATCH, EMBED), jnp.float32),
    scratch_types=[
        pltpu.VMEM((NB, CTX), jnp.int32),
        pltpu.VMEM((NIDX, EMBED), jnp.float32),
        pltpu.VMEM((NB, EMBED), jnp.float32),
        pltpu.SemaphoreType.DMA,
    ],
    compiler_params=pltpu.CompilerParams(use_tc_tiling_on_sc=False),
)
def _pool_sc(idx_hbm, table_hbm, out_hbm, idx_v, rows_v, acc_v, sem):
    wid = lax.axis_index("s") * NC + lax.axis_index("c")
    base_b = wid * NB

    pltpu.sync_copy(idx_hbm.at[pl.ds(base_b, NB)], idx_v)

    copies = []
    for r in range(NB):
        copies.append(
            pltpu.async_copy(
                table_hbm.at[idx_v.at[r]],
                rows_v.at[pl.ds(r * CTX, CTX)],
                sem,
            )
        )
    for cp in copies:
        cp.wait()

    scale = jnp.float32(1.0 / CTX)

    def batch_body(b, carry):
        r0 = b * CTX

        def c_body(c, accs):
            a0, a1, a2, a3 = accs
            r = r0 + c
            a0 = a0 + rows_v[r, pl.ds(0, 16)]
            a1 = a1 + rows_v[r, pl.ds(16, 16)]
            a2 = a2 + rows_v[r, pl.ds(32, 16)]
            a3 = a3 + rows_v[r, pl.ds(48, 16)]
            return (a0, a1, a2, a3)

        z = jnp.zeros((16,), jnp.float32)
        a0, a1, a2, a3 = lax.fori_loop(0, CTX, c_body, (z, z, z, z))
        acc_v[b, pl.ds(0, 16)] = a0 * scale
        acc_v[b, pl.ds(16, 16)] = a1 * scale
        acc_v[b, pl.ds(32, 16)] = a2 * scale
        acc_v[b, pl.ds(48, 16)] = a3 * scale
        return carry

    lax.fori_loop(0, NB, batch_body, 0)

    pltpu.sync_copy(acc_v, out_hbm.at[pl.ds(base_b, NB)])


VB = 2048
VGRID = (VOCAB + VB - 1) // VB


def _mm_body(x_ref, w_ref, b_ref, o_ref):
    acc = lax.dot_general(
        w_ref[...].astype(jnp.bfloat16),
        x_ref[...].astype(jnp.bfloat16),
        dimension_numbers=(((0,), (1,)), ((), ())),
        preferred_element_type=jnp.float32,
    )
    o_ref[...] = acc + jnp.transpose(b_ref[...], (1, 0))


def _dense_tc(x, W, b2d):
    out_t = pl.pallas_call(
        _mm_body,
        grid=(VGRID,),
        in_specs=[
            pl.BlockSpec((BATCH, EMBED), lambda i: (0, 0)),
            pl.BlockSpec((EMBED, VB), lambda i: (0, i)),
            pl.BlockSpec((1, VB), lambda i: (0, i)),
        ],
        out_specs=pl.BlockSpec((VB, BATCH), lambda i: (i, 0)),
        out_shape=jax.ShapeDtypeStruct((VOCAB, BATCH), jnp.float32),
        compiler_params=pltpu.CompilerParams(
            dimension_semantics=("arbitrary",),
        ),
    )(x, W, b2d)
    return out_t.T


def kernel(inputs, emb_table, W, b):
    pooled = _pool_sc(inputs.astype(jnp.int32), emb_table)
    return _dense_tc(pooled, W, b.reshape(1, VOCAB))


# speedup vs baseline: 1.0057x; 1.0057x over previous
import functools

import jax
import jax.numpy as jnp
from jax import lax
from jax.experimental import pallas as pl
from jax.experimental.pallas import tpu as pltpu
from jax.experimental.pallas import tpu_sc as plsc

VOCAB = 100000
EMBED = 64
BATCH = 1024
CTX = 50

NC = 2
NS = 16
NW = NC * NS
NB = BATCH // NW
NIDX = NB * CTX

_sc_mesh = plsc.VectorSubcoreMesh(core_axis_name="c", subcore_axis_name="s")


@functools.partial(
    pl.kernel,
    mesh=_sc_mesh,
    out_type=jax.ShapeDtypeStruct((BATCH, EMBED), jnp.float32),
    scratch_types=[
        pltpu.VMEM((CTX, NB), jnp.int32),
        pltpu.VMEM((NIDX, EMBED), jnp.float32),
        pltpu.VMEM((NB, EMBED), jnp.float32),
        pltpu.SemaphoreType.DMA,
    ],
    compiler_params=pltpu.CompilerParams(use_tc_tiling_on_sc=False),
)
def _pool_sc(idx_hbm, table_hbm, out_hbm, idx_v, rows_v, acc_v, sem):
    wid = lax.axis_index("s") * NC + lax.axis_index("c")
    base_b = wid * NB

    pltpu.sync_copy(idx_hbm.at[:, pl.ds(base_b, NB)], idx_v)

    def fire(c, carry):
        pltpu.async_copy(
            table_hbm.at[idx_v.at[c]],
            rows_v.at[pl.ds(c * NB, NB)],
            sem,
        )
        return carry

    lax.fori_loop(0, CTX, fire, 0)

    def drain(c, carry):
        pltpu.make_async_copy(
            table_hbm.at[idx_v.at[0]],
            rows_v.at[pl.ds(0, NB)],
            sem,
        ).wait()
        return carry

    lax.fori_loop(0, CTX, drain, 0)

    scale = jnp.float32(1.0 / CTX)

    def batch_body(b, carry):
        def c_body(c, accs):
            a0, a1, a2, a3 = accs
            r = c * NB + b
            a0 = a0 + rows_v[r, pl.ds(0, 16)]
            a1 = a1 + rows_v[r, pl.ds(16, 16)]
            a2 = a2 + rows_v[r, pl.ds(32, 16)]
            a3 = a3 + rows_v[r, pl.ds(48, 16)]
            return (a0, a1, a2, a3)

        z = jnp.zeros((16,), jnp.float32)
        a0, a1, a2, a3 = lax.fori_loop(0, CTX, c_body, (z, z, z, z))
        acc_v[b, pl.ds(0, 16)] = a0 * scale
        acc_v[b, pl.ds(16, 16)] = a1 * scale
        acc_v[b, pl.ds(32, 16)] = a2 * scale
        acc_v[b, pl.ds(48, 16)] = a3 * scale
        return carry

    lax.fori_loop(0, NB, batch_body, 0)

    pltpu.sync_copy(acc_v, out_hbm.at[pl.ds(base_b, NB)])


VB = 2048
VGRID = (VOCAB + VB - 1) // VB


def _mm_body(x_ref, w_ref, b_ref, o_ref):
    acc = lax.dot_general(
        w_ref[...].astype(jnp.bfloat16),
        x_ref[...].astype(jnp.bfloat16),
        dimension_numbers=(((0,), (1,)), ((), ())),
        preferred_element_type=jnp.float32,
    )
    o_ref[...] = acc + jnp.transpose(b_ref[...], (1, 0))


def _dense_tc(x, W, b2d):
    out_t = pl.pallas_call(
        _mm_body,
        grid=(VGRID,),
        in_specs=[
            pl.BlockSpec((BATCH, EMBED), lambda i: (0, 0)),
            pl.BlockSpec((EMBED, VB), lambda i: (0, i)),
            pl.BlockSpec((1, VB), lambda i: (0, i)),
        ],
        out_specs=pl.BlockSpec((VB, BATCH), lambda i: (i, 0)),
        out_shape=jax.ShapeDtypeStruct((VOCAB, BATCH), jnp.float32),
        compiler_params=pltpu.CompilerParams(
            dimension_semantics=("arbitrary",),
        ),
    )(x, W, b2d)
    return out_t.T


def kernel(inputs, emb_table, W, b):
    pooled = _pool_sc(inputs.astype(jnp.int32).T, emb_table)
    return _dense_tc(pooled, W, b.reshape(1, VOCAB))


# speedup vs baseline: 1.0150x; 1.0092x over previous
import functools

import jax
import jax.numpy as jnp
from jax import lax
from jax.experimental import pallas as pl
from jax.experimental.pallas import tpu as pltpu
from jax.experimental.pallas import tpu_sc as plsc

VOCAB = 100000
EMBED = 64
BATCH = 1024
CTX = 50

NC = 2
NS = 16
NW = NC * NS
NB = BATCH // NW
NIDX = NB * CTX

_sc_mesh = plsc.VectorSubcoreMesh(core_axis_name="c", subcore_axis_name="s")


@functools.partial(
    pl.kernel,
    mesh=_sc_mesh,
    out_type=jax.ShapeDtypeStruct((BATCH, EMBED), jnp.float32),
    scratch_types=[
        pltpu.VMEM((CTX, NB), jnp.int32),
        pltpu.VMEM((NIDX, EMBED), jnp.float32),
        pltpu.VMEM((NB, EMBED), jnp.float32),
        pltpu.SemaphoreType.DMA,
    ],
    compiler_params=pltpu.CompilerParams(use_tc_tiling_on_sc=False),
)
def _pool_sc(idx_hbm, table_hbm, out_hbm, idx_v, rows_v, acc_v, sem):
    wid = lax.axis_index("s") * NC + lax.axis_index("c")
    base_b = wid * NB

    pltpu.sync_copy(idx_hbm.at[:, pl.ds(base_b, NB)], idx_v)

    def fire(c, carry):
        pltpu.async_copy(
            table_hbm.at[idx_v.at[c]],
            rows_v.at[pl.ds(c * NB, NB)],
            sem,
        )
        return carry

    lax.fori_loop(0, CTX, fire, 0)

    def drain(c, carry):
        pltpu.make_async_copy(
            table_hbm.at[idx_v.at[0]],
            rows_v.at[pl.ds(0, NB)],
            sem,
        ).wait()
        return carry

    lax.fori_loop(0, CTX, drain, 0)

    scale = jnp.float32(1.0 / CTX)

    def batch_body(b, carry):
        def c_body(c, accs):
            a0, a1, a2, a3 = accs
            r = c * NB + b
            a0 = a0 + rows_v[r, pl.ds(0, 16)]
            a1 = a1 + rows_v[r, pl.ds(16, 16)]
            a2 = a2 + rows_v[r, pl.ds(32, 16)]
            a3 = a3 + rows_v[r, pl.ds(48, 16)]
            return (a0, a1, a2, a3)

        z = jnp.zeros((16,), jnp.float32)
        a0, a1, a2, a3 = lax.fori_loop(0, CTX, c_body, (z, z, z, z))
        acc_v[b, pl.ds(0, 16)] = a0 * scale
        acc_v[b, pl.ds(16, 16)] = a1 * scale
        acc_v[b, pl.ds(32, 16)] = a2 * scale
        acc_v[b, pl.ds(48, 16)] = a3 * scale
        return carry

    lax.fori_loop(0, NB, batch_body, 0)

    pltpu.sync_copy(acc_v, out_hbm.at[pl.ds(base_b, NB)])


VB = 4096
VGRID = (VOCAB + VB - 1) // VB


def _mm_body(x_ref, w_ref, b_ref, o_ref):
    acc = lax.dot_general(
        w_ref[...].astype(jnp.bfloat16),
        x_ref[...].astype(jnp.bfloat16),
        dimension_numbers=(((0,), (1,)), ((), ())),
        preferred_element_type=jnp.float32,
    )
    o_ref[...] = acc + jnp.transpose(b_ref[...], (1, 0))


def _dense_tc(x, W, b2d):
    out_t = pl.pallas_call(
        _mm_body,
        grid=(VGRID,),
        in_specs=[
            pl.BlockSpec((BATCH, EMBED), lambda i: (0, 0)),
            pl.BlockSpec((EMBED, VB), lambda i: (0, i)),
            pl.BlockSpec((1, VB), lambda i: (0, i)),
        ],
        out_specs=pl.BlockSpec((VB, BATCH), lambda i: (i, 0)),
        out_shape=jax.ShapeDtypeStruct((VOCAB, BATCH), jnp.float32),
        compiler_params=pltpu.CompilerParams(
            dimension_semantics=("arbitrary",),
        ),
    )(x, W, b2d)
    return out_t.T


def kernel(inputs, emb_table, W, b):
    pooled = _pool_sc(inputs.astype(jnp.int32).T, emb_table)
    return _dense_tc(pooled, W, b.reshape(1, VOCAB))
